# baseline (speedup 1.0000x reference)
import functools

import jax
import jax.numpy as jnp
import numpy as np
from jax import lax
from jax.experimental import pallas as pl
from jax.experimental.pallas import tpu as pltpu
from jax.experimental.pallas import tpu_sc as plsc

B, S, D, H, HD, E, K, F, V, L = 1, 2048, 1024, 16, 64, 16, 2, 1024, 32000, 2
ALPHA, LIMIT, EPS = 1.702, 7.0, 1e-5
T = B * S
C = (T * K * 3) // (E * 2)
TK = T * K

_NC, _NS = 2, 16
_NW = _NC * _NS
_CHUNK = 64


def _sc_gather(table, idx):
  nrows = idx.shape[0]
  dcols = table.shape[1]
  n_chunks = nrows // (_NW * _CHUNK)
  assert nrows == n_chunks * _NW * _CHUNK
  mesh = plsc.VectorSubcoreMesh(core_axis_name="c", subcore_axis_name="s")

  @functools.partial(
      pl.kernel, mesh=mesh,
      out_type=jax.ShapeDtypeStruct((nrows, dcols), jnp.float32),
      scratch_types=[
          pltpu.VMEM((_CHUNK,), jnp.int32),
          pltpu.VMEM((_CHUNK, dcols), jnp.float32),
          pltpu.SemaphoreType.DMA,
      ],
  )
  def gk(table_hbm, idx_hbm, out_hbm, idx_v, rows_v, sem):
    wid = lax.axis_index("s") * _NC + lax.axis_index("c")
    for j in range(n_chunks):
      base = (wid * n_chunks + j) * _CHUNK
      pltpu.sync_copy(idx_hbm.at[pl.ds(base, _CHUNK)], idx_v)
      pltpu.async_copy(table_hbm.at[idx_v], rows_v, sem).wait()
      pltpu.sync_copy(rows_v, out_hbm.at[pl.ds(base, _CHUNK)])

  return gk(table, idx)


_SLOT_PAD = 8


def _sc_build_src(dst_safe, tvals, zeros):
  nslots = E * C + _SLOT_PAD
  mesh = plsc.VectorSubcoreMesh(core_axis_name="c", subcore_axis_name="s")

  @functools.partial(
      pl.kernel, mesh=mesh,
      out_type=jax.ShapeDtypeStruct((nslots,), jnp.int32),
      scratch_types=[
          pltpu.VMEM((TK,), jnp.int32),
          pltpu.VMEM((TK,), jnp.int32),
          pltpu.VMEM((nslots,), jnp.int32),
          pltpu.SemaphoreType.DMA,
      ],
  )
  def sk(dst_hbm, tv_hbm, zero_hbm, out_hbm, dst_v, tv_v, z_v, sem):
    wid = lax.axis_index("s") * _NC + lax.axis_index("c")

    @pl.when(wid == 0)
    def _():
      pltpu.sync_copy(zero_hbm, z_v)
      pltpu.sync_copy(z_v, out_hbm)
      pltpu.sync_copy(dst_hbm, dst_v)
      pltpu.sync_copy(tv_hbm, tv_v)
      pltpu.async_copy(tv_v, out_hbm.at[dst_v], sem).wait()

  return sk(dst_safe, tvals, zeros)


_RB = 512
_NRB = S // _RB


def _rms(x, w):
  return x * lax.rsqrt(jnp.mean(x * x, axis=-1, keepdims=True) + EPS) * w


def _dot(a, b):
  return lax.dot(a.astype(jnp.bfloat16), b.astype(jnp.bfloat16),
                 preferred_element_type=jnp.float32)


def _dot_t(a, b):
  return lax.dot_general(a.astype(jnp.bfloat16), b.astype(jnp.bfloat16),
                         (((1,), (1,)), ((), ())),
                         preferred_element_type=jnp.float32)


def _rope(x, cs, sn):
  half = HD // 2
  x1 = x[:, :half]
  x2 = x[:, half:]
  rot = jnp.concatenate([-x2, x1], axis=-1)
  return x * cs + rot * sn


def _attn_call(x, ln1, wq3, bq3, wk3, bk3, wv3, bv3, cs, sn):
  scale = 1.0 / np.sqrt(HD)

  def body(x_ref, ln_ref, wq_ref, bq_ref, wk_ref, bk_ref, wv_ref, bv_ref,
           cs_ref, sn_ref, o_ref, h1_s, k_s, v_s):
    h = pl.program_id(0)
    rb = pl.program_id(1)

    @pl.when((h == 0) & (rb == 0))
    def _():
      h1_s[...] = _rms(x_ref[...], ln_ref[...])

    @pl.when(rb == 0)
    def _():
      h1 = h1_s[...]
      k_s[...] = _rope(_dot(h1, wk_ref[...]) + bk_ref[...],
                       cs_ref[...], sn_ref[...])
      v_s[...] = _dot(h1, wv_ref[...]) + bv_ref[...]

    rows = pl.ds(rb * _RB, _RB)
    q = _rope(_dot(h1_s[rows, :], wq_ref[...]) + bq_ref[...],
              cs_ref[rows, :], sn_ref[rows, :])
    att = _dot_t(q, k_s[...]) * scale
    row = rb * _RB + lax.broadcasted_iota(jnp.int32, (_RB, S), 0)
    col = lax.broadcasted_iota(jnp.int32, (_RB, S), 1)
    att = jnp.where(row >= col, att, -1e30)
    m = jnp.max(att, axis=-1, keepdims=True)
    p = jnp.exp(att - m)
    p = p / jnp.sum(p, axis=-1, keepdims=True)
    o_ref[...] = _dot(p, v_s[...])

  return pl.pallas_call(
      body,
      grid=(H, _NRB),
      in_specs=[
          pl.BlockSpec((S, D), lambda h, r: (0, 0)),
          pl.BlockSpec((1, D), lambda h, r: (0, 0)),
          pl.BlockSpec((None, D, HD), lambda h, r: (h, 0, 0)),
          pl.BlockSpec((None, 1, HD), lambda h, r: (h, 0, 0)),
          pl.BlockSpec((None, D, HD), lambda h, r: (h, 0, 0)),
          pl.BlockSpec((None, 1, HD), lambda h, r: (h, 0, 0)),
          pl.BlockSpec((None, D, HD), lambda h, r: (h, 0, 0)),
          pl.BlockSpec((None, 1, HD), lambda h, r: (h, 0, 0)),
          pl.BlockSpec((S, HD), lambda h, r: (0, 0)),
          pl.BlockSpec((S, HD), lambda h, r: (0, 0)),
      ],
      out_specs=pl.BlockSpec((None, _RB, HD), lambda h, r: (h, r, 0)),
      out_shape=jax.ShapeDtypeStruct((H, S, HD), jnp.float32),
      scratch_shapes=[
          pltpu.VMEM((S, D), jnp.float32),
          pltpu.VMEM((S, HD), jnp.float32),
          pltpu.VMEM((S, HD), jnp.float32),
      ],
  )(x, ln1, wq3, bq3, wk3, bk3, wv3, bv3, cs, sn)


def _oproj_call(x, o, wo, bo):
  def body(x_ref, o_ref, w_ref, b_ref, out_ref):
    out_ref[...] = x_ref[...] + _dot(o_ref[...], w_ref[...]) + b_ref[...]

  return pl.pallas_call(
      body,
      grid=(_NRB,),
      in_specs=[
          pl.BlockSpec((_RB, D), lambda r: (r, 0)),
          pl.BlockSpec((_RB, D), lambda r: (r, 0)),
          pl.BlockSpec((D, D), lambda r: (0, 0)),
          pl.BlockSpec((1, D), lambda r: (0, 0)),
      ],
      out_specs=pl.BlockSpec((_RB, D), lambda r: (r, 0)),
      out_shape=jax.ShapeDtypeStruct((S, D), jnp.float32),
  )(x, o, wo, bo)


def _router_call(x, ln2, wg, bg):
  def body(x_ref, ln_ref, wg_ref, bg_ref,
           h2_ref, g1_ref, i1_ref, g2_ref, i2_ref, ab_ref):
    h2 = _rms(x_ref[...], ln_ref[...])
    h2_ref[...] = h2
    logits = _dot(h2, wg_ref[...]) + bg_ref[...]
    m = jnp.max(logits, axis=-1, keepdims=True)
    ex = jnp.exp(logits - m)
    probs = ex / jnp.sum(ex, axis=-1, keepdims=True)
    lanes = lax.broadcasted_iota(jnp.int32, (_RB, E), 1)
    g1 = jnp.max(probs, axis=-1, keepdims=True)
    i1 = jnp.min(jnp.where(probs >= g1, lanes, E), axis=-1, keepdims=True)
    oh1 = lanes == i1
    masked = jnp.where(oh1, -1.0, probs)
    g2 = jnp.max(masked, axis=-1, keepdims=True)
    i2 = jnp.min(jnp.where(masked >= g2, lanes, E), axis=-1, keepdims=True)
    oh2 = lanes == i2
    g1_ref[...] = g1
    i1_ref[...] = i1
    g2_ref[...] = g2
    i2_ref[...] = i2
    ab_ref[...] = oh1.astype(jnp.float32) + oh2.astype(jnp.float32)

  return pl.pallas_call(
      body,
      grid=(_NRB,),
      in_specs=[
          pl.BlockSpec((_RB, D), lambda r: (r, 0)),
          pl.BlockSpec((1, D), lambda r: (0, 0)),
          pl.BlockSpec((D, E), lambda r: (0, 0)),
          pl.BlockSpec((1, E), lambda r: (0, 0)),
      ],
      out_specs=[
          pl.BlockSpec((_RB, D), lambda r: (r, 0)),
          pl.BlockSpec((_RB, 1), lambda r: (r, 0)),
          pl.BlockSpec((_RB, 1), lambda r: (r, 0)),
          pl.BlockSpec((_RB, 1), lambda r: (r, 0)),
          pl.BlockSpec((_RB, 1), lambda r: (r, 0)),
          pl.BlockSpec((_RB, E), lambda r: (r, 0)),
      ],
      out_shape=[
          jax.ShapeDtypeStruct((S, D), jnp.float32),
          jax.ShapeDtypeStruct((S, 1), jnp.float32),
          jax.ShapeDtypeStruct((S, 1), jnp.int32),
          jax.ShapeDtypeStruct((S, 1), jnp.float32),
          jax.ShapeDtypeStruct((S, 1), jnp.int32),
          jax.ShapeDtypeStruct((S, E), jnp.float32),
      ],
  )(x, ln2, wg, bg)


_PB = 256


def _posn_call(ab, i1, i2, g1, g2):
  def body(ab_ref, i1_ref, i2_ref, g1_ref, g2_ref,
           dst_ref, keep_ref, w_ref, carry):
    @pl.when(pl.program_id(0) == 0)
    def _():
      carry[...] = jnp.zeros((8, E), jnp.float32)

    c = carry[0:1, :]
    abb = ab_ref[...]
    rid = lax.broadcasted_iota(jnp.int32, (_PB, _PB), 0)
    cid = lax.broadcasted_iota(jnp.int32, (_PB, _PB), 1)
    tri = (rid > cid).astype(jnp.float32)
    p = c + lax.dot(tri, abb)
    carry[...] = jnp.broadcast_to(c + jnp.sum(abb, axis=0, keepdims=True),
                                  (8, E))
    lanes = lax.broadcasted_iota(jnp.int32, (_PB, E), 1)
    oh1 = (lanes == i1_ref[...]).astype(jnp.float32)
    oh2 = (lanes == i2_ref[...]).astype(jnp.float32)
    pos1 = jnp.sum(p * oh1, axis=-1, keepdims=True)
    pos2 = jnp.sum(p * oh2, axis=-1, keepdims=True)
    keep1 = pos1 < C
    keep2 = pos2 < C
    p1 = jnp.minimum(pos1, C - 1).astype(jnp.int32)
    p2 = jnp.minimum(pos2, C - 1).astype(jnp.int32)
    d1 = i1_ref[...] * C + p1
    d2 = i2_ref[...] * C + p2
    dst_ref[...] = jnp.concatenate([d1, d2], axis=1)
    keep_ref[...] = jnp.concatenate(
        [jnp.where(keep1, d1, E * C), jnp.where(keep2, d2, E * C)], axis=1)
    w_ref[...] = jnp.concatenate(
        [jnp.where(keep1, g1_ref[...], 0.0),
         jnp.where(keep2, g2_ref[...], 0.0)], axis=1)

  return pl.pallas_call(
      body,
      grid=(S // _PB,),
      in_specs=[
          pl.BlockSpec((_PB, E), lambda r: (r, 0)),
          pl.BlockSpec((_PB, 1), lambda r: (r, 0)),
          pl.BlockSpec((_PB, 1), lambda r: (r, 0)),
          pl.BlockSpec((_PB, 1), lambda r: (r, 0)),
          pl.BlockSpec((_PB, 1), lambda r: (r, 0)),
      ],
      out_specs=[
          pl.BlockSpec((_PB, 2), lambda r: (r, 0)),
          pl.BlockSpec((_PB, 2), lambda r: (r, 0)),
          pl.BlockSpec((_PB, 2), lambda r: (r, 0)),
      ],
      out_shape=[
          jax.ShapeDtypeStruct((S, 2), jnp.int32),
          jax.ShapeDtypeStruct((S, 2), jnp.int32),
          jax.ShapeDtypeStruct((S, 2), jnp.float32),
      ],
      scratch_shapes=[pltpu.VMEM((8, E), jnp.float32)],
  )(ab, i1, i2, g1, g2)


def _ffn_call(xin, w1, b1, w2, b2):
  def body(x_ref, w1_ref, b1_ref, w2_ref, b2_ref, o_ref):
    gu = _dot(x_ref[...], w1_ref[...]) + b1_ref[...]
    g = jnp.minimum(gu[:, :F], LIMIT)
    u = jnp.clip(gu[:, F:], -LIMIT, LIMIT)
    sig = 1.0 / (1.0 + jnp.exp(-ALPHA * g))
    act = g * sig * (u + 1.0)
    o_ref[...] = _dot(act, w2_ref[...]) + b2_ref[...]

  return pl.pallas_call(
      body,
      grid=(E,),
      in_specs=[
          pl.BlockSpec((None, C, D), lambda e: (e, 0, 0)),
          pl.BlockSpec((None, D, 2 * F), lambda e: (e, 0, 0)),
          pl.BlockSpec((None, 1, 2 * F), lambda e: (e, 0, 0)),
          pl.BlockSpec((None, F, D), lambda e: (e, 0, 0)),
          pl.BlockSpec((None, 1, D), lambda e: (e, 0, 0)),
      ],
      out_specs=pl.BlockSpec((None, C, D), lambda e: (e, 0, 0)),
      out_shape=jax.ShapeDtypeStruct((E, C, D), jnp.float32),
  )(xin, w1, b1, w2, b2)


def _combine_call(x, rcat, w):
  def body(x_ref, r_ref, w_ref, o_ref):
    wv = w_ref[...]
    o_ref[...] = (x_ref[...]
                  + wv[:, 0:1] * r_ref[:, :D]
                  + wv[:, 1:2] * r_ref[:, D:])

  return pl.pallas_call(
      body,
      grid=(_NRB,),
      in_specs=[
          pl.BlockSpec((_RB, D), lambda r: (r, 0)),
          pl.BlockSpec((_RB, 2 * D), lambda r: (r, 0)),
          pl.BlockSpec((_RB, 2), lambda r: (r, 0)),
      ],
      out_specs=pl.BlockSpec((_RB, D), lambda r: (r, 0)),
      out_shape=jax.ShapeDtypeStruct((S, D), jnp.float32),
  )(x, rcat, w)


def _final_call(x, normf):
  def body(x_ref, w_ref, o_ref):
    o_ref[...] = _rms(x_ref[...], w_ref[...])

  return pl.pallas_call(
      body,
      grid=(_NRB,),
      in_specs=[
          pl.BlockSpec((_RB, D), lambda r: (r, 0)),
          pl.BlockSpec((1, D), lambda r: (0, 0)),
      ],
      out_specs=pl.BlockSpec((_RB, D), lambda r: (r, 0)),
      out_shape=jax.ShapeDtypeStruct((S, D), jnp.float32),
  )(x, normf)


def kernel(params, input_ids):
  ids = input_ids.reshape(S).astype(jnp.int32)
  x = _sc_gather(params['embed'], ids)

  pos = jnp.arange(S, dtype=jnp.float32)
  inv = 1.0 / (10000.0 ** (jnp.arange(0, HD, 2, dtype=jnp.float32) / HD))
  ang = pos[:, None] * inv[None, :]
  cs = jnp.concatenate([jnp.cos(ang), jnp.cos(ang)], axis=-1)
  sn = jnp.concatenate([jnp.sin(ang), jnp.sin(ang)], axis=-1)
  zeros_slots = jnp.zeros((E * C + _SLOT_PAD,), jnp.int32)
  tvals = (jnp.arange(TK, dtype=jnp.int32) // K).astype(jnp.int32)

  for lp in params['layers']:
    w3 = lambda w: w.reshape(D, H, HD).transpose(1, 0, 2)
    b3 = lambda b: b.reshape(H, 1, HD)
    oh = _attn_call(x, lp['ln1'][None],
                    w3(lp['Wq']), b3(lp['bq']),
                    w3(lp['Wk']), b3(lp['bk']),
                    w3(lp['Wv']), b3(lp['bv']), cs, sn)
    o = oh.transpose(1, 0, 2).reshape(S, D)
    x = _oproj_call(x, o, lp['Wo'], lp['bo'][None])

    h2, g1, i1, g2, i2, ab = _router_call(x, lp['ln2'][None], lp['Wg'],
                                          lp['bg'][None])
    dst, dst_safe, w = _posn_call(ab, i1, i2, g1, g2)
    dst_flat = dst.reshape(TK)
    src = _sc_build_src(dst_safe.reshape(TK), tvals, zeros_slots)
    xin = _sc_gather(h2, src[:E * C])
    eo = _ffn_call(xin.reshape(E, C, D), lp['W1'], lp['b1'][:, None, :],
                   lp['W2'], lp['b2'][:, None, :])
    rows = _sc_gather(eo.reshape(E * C, D), dst_flat)
    x = _combine_call(x, rows.reshape(S, 2 * D), w)

  out = _final_call(x, params['normf'][None])
  return out.reshape(B, S, D)
